# baseline (speedup 1.0000x reference)
import jax
import jax.numpy as jnp
from jax import lax
from jax.experimental import pallas as pl

HCT = 2048
TEXT_LEN = 4096
VIS_LEN = 1024
AUD_LEN = 1024
TOTAL = TEXT_LEN + VIS_LEN + AUD_LEN
VOCAB = 100000
NEXP = 16

ROW_CHUNK = 512
N_TEXT_CHUNKS = TEXT_LEN // ROW_CHUNK
V_TILE = 512


def _mean_body(text_ref, vis_ref, aud_ref, rw_ref, out_ref, top_ref):
    step = pl.program_id(0)
    part = jnp.sum(text_ref[...], axis=0, keepdims=True)

    @pl.when(step == 0)
    def _init():
        rest = (jnp.sum(vis_ref[...], axis=0, keepdims=True)
                + jnp.sum(aud_ref[...], axis=0, keepdims=True))
        out_ref[...] = part + rest

    @pl.when(step != 0)
    def _acc():
        out_ref[...] += part

    @pl.when(step == N_TEXT_CHUNKS - 1)
    def _fin():
        fused = out_ref[...] * (1.0 / TOTAL)
        out_ref[...] = fused
        logits = lax.dot_general(
            fused, rw_ref[...],
            dimension_numbers=(((1,), (1,)), ((), ())),
            preferred_element_type=jnp.float32,
        )
        a0 = jnp.argmax(logits, axis=1)[0]
        cols = lax.broadcasted_iota(jnp.int32, (1, NEXP), 1)
        masked = jnp.where(cols == a0, -jnp.inf, logits)
        a1 = jnp.argmax(masked, axis=1)[0]
        out2 = lax.broadcasted_iota(jnp.int32, (1, 2), 1)
        top_ref[...] = jnp.where(out2 == 0, a0.astype(jnp.int32), a1.astype(jnp.int32))


def _fused_mean(text, vis, aud, rw):
    return pl.pallas_call(
        _mean_body,
        grid=(N_TEXT_CHUNKS,),
        in_specs=[
            pl.BlockSpec((ROW_CHUNK, HCT), lambda i: (i, 0)),
            pl.BlockSpec((VIS_LEN, HCT), lambda i: (0, 0)),
            pl.BlockSpec((AUD_LEN, HCT), lambda i: (0, 0)),
            pl.BlockSpec((NEXP, HCT), lambda i: (0, 0)),
        ],
        out_specs=[
            pl.BlockSpec((1, HCT), lambda i: (0, 0)),
            pl.BlockSpec((1, 2), lambda i: (0, 0)),
        ],
        out_shape=[
            jax.ShapeDtypeStruct((1, HCT), jnp.float32),
            jax.ShapeDtypeStruct((1, 2), jnp.int32),
        ],
    )(text, vis, aud, rw)


def _proj_body(fused_ref, w_ref, b_ref, out_ref):
    acc = lax.dot_general(
        fused_ref[...], w_ref[...],
        dimension_numbers=(((1,), (1,)), ((), ())),
        preferred_element_type=jnp.float32,
    )
    out_ref[...] = acc + b_ref[...]


def _vocab_proj(fused, w, b):
    n_tiles = pl.cdiv(VOCAB, V_TILE)
    return pl.pallas_call(
        _proj_body,
        grid=(n_tiles,),
        in_specs=[
            pl.BlockSpec((1, HCT), lambda i: (0, 0)),
            pl.BlockSpec((V_TILE, HCT), lambda i: (i, 0)),
            pl.BlockSpec((1, V_TILE), lambda i: (0, i)),
        ],
        out_specs=pl.BlockSpec((1, V_TILE), lambda i: (0, i)),
        out_shape=jax.ShapeDtypeStruct((1, VOCAB), jnp.float32),
    )(fused, w, b)


def kernel(text_vector, visual_vector, audio_vector, router_weight, output_weight, output_bias):
    fused, topk = _fused_mean(text_vector, visual_vector, audio_vector, router_weight)
    logits = _vocab_proj(fused, output_weight, output_bias[None, :])
    return (logits, topk)


# speedup vs baseline: 1.2116x; 1.2116x over previous
import jax
import jax.numpy as jnp
from jax import lax
from jax.experimental import pallas as pl

HCT = 2048
TEXT_LEN = 4096
VIS_LEN = 1024
AUD_LEN = 1024
TOTAL = TEXT_LEN + VIS_LEN + AUD_LEN
VOCAB = 100000
NEXP = 16

ROW_CHUNK = 1024
N_TEXT_CHUNKS = TEXT_LEN // ROW_CHUNK
V_TILE = 1024


def _mean_body(text_ref, vis_ref, aud_ref, rw_ref, out_ref, top_ref):
    step = pl.program_id(0)
    part = jnp.sum(text_ref[...], axis=0, keepdims=True)

    @pl.when(step == 0)
    def _init():
        rest = (jnp.sum(vis_ref[...], axis=0, keepdims=True)
                + jnp.sum(aud_ref[...], axis=0, keepdims=True))
        out_ref[...] = part + rest

    @pl.when(step != 0)
    def _acc():
        out_ref[...] += part

    @pl.when(step == N_TEXT_CHUNKS - 1)
    def _fin():
        fused = out_ref[...] * (1.0 / TOTAL)
        out_ref[...] = fused
        logits = lax.dot_general(
            fused, rw_ref[...],
            dimension_numbers=(((1,), (1,)), ((), ())),
            preferred_element_type=jnp.float32,
        )
        a0 = jnp.argmax(logits, axis=1)[0]
        cols = lax.broadcasted_iota(jnp.int32, (1, NEXP), 1)
        masked = jnp.where(cols == a0, -jnp.inf, logits)
        a1 = jnp.argmax(masked, axis=1)[0]
        out2 = lax.broadcasted_iota(jnp.int32, (1, 2), 1)
        top_ref[...] = jnp.where(out2 == 0, a0.astype(jnp.int32), a1.astype(jnp.int32))


def _fused_mean(text, vis, aud, rw):
    return pl.pallas_call(
        _mean_body,
        grid=(N_TEXT_CHUNKS,),
        in_specs=[
            pl.BlockSpec((ROW_CHUNK, HCT), lambda i: (i, 0)),
            pl.BlockSpec((VIS_LEN, HCT), lambda i: (0, 0)),
            pl.BlockSpec((AUD_LEN, HCT), lambda i: (0, 0)),
            pl.BlockSpec((NEXP, HCT), lambda i: (0, 0)),
        ],
        out_specs=[
            pl.BlockSpec((1, HCT), lambda i: (0, 0)),
            pl.BlockSpec((1, 2), lambda i: (0, 0)),
        ],
        out_shape=[
            jax.ShapeDtypeStruct((1, HCT), jnp.float32),
            jax.ShapeDtypeStruct((1, 2), jnp.int32),
        ],
    )(text, vis, aud, rw)


def _proj_body(fused_ref, w_ref, b_ref, out_ref):
    acc = lax.dot_general(
        fused_ref[...], w_ref[...],
        dimension_numbers=(((1,), (1,)), ((), ())),
        preferred_element_type=jnp.float32,
    )
    out_ref[...] = acc + b_ref[...]


def _vocab_proj(fused, w, b):
    n_tiles = pl.cdiv(VOCAB, V_TILE)
    return pl.pallas_call(
        _proj_body,
        grid=(n_tiles,),
        in_specs=[
            pl.BlockSpec((1, HCT), lambda i: (0, 0)),
            pl.BlockSpec((V_TILE, HCT), lambda i: (i, 0)),
            pl.BlockSpec((1, V_TILE), lambda i: (0, i)),
        ],
        out_specs=pl.BlockSpec((1, V_TILE), lambda i: (0, i)),
        out_shape=jax.ShapeDtypeStruct((1, VOCAB), jnp.float32),
    )(fused, w, b)


def kernel(text_vector, visual_vector, audio_vector, router_weight, output_weight, output_bias):
    fused, topk = _fused_mean(text_vector, visual_vector, audio_vector, router_weight)
    logits = _vocab_proj(fused, output_weight, output_bias[None, :])
    return (logits, topk)
